# baseline (speedup 1.0000x reference)
import functools

import jax
import jax.numpy as jnp
from jax import lax
from jax.experimental import pallas as pl
from jax.experimental.pallas import tpu as pltpu
from jax.experimental.pallas import tpu_sc as plsc

N, C, H, W = 4, 96, 384, 384
CP = 128
HW = H * W
NP = N * HW
NC, NS, L = 2, 16, 16
NW = NC * NS
PXW = NP // NW
P = 64
CHUNKS = PXW // P
GB = 24
GBP = GB * P
PP = P + 1


def _sc_grid_sample(xt, gxy):
    mesh = plsc.VectorSubcoreMesh(
        core_axis_name="c", subcore_axis_name="s", num_cores=NC,
        num_subcores=NS)

    scratch = (
        [pltpu.VMEM((2, GBP), jnp.float32)]
        + [pltpu.VMEM((P,), jnp.int32)] * 8
        + [pltpu.VMEM((P,), jnp.float32)] * 8
        + [pltpu.VMEM((P, CP), jnp.float32)] * 8
        + [pltpu.VMEM((C, PP), jnp.float32)] * 2
        + [pltpu.SemaphoreType.DMA] * 4
    )

    @functools.partial(
        pl.kernel,
        out_type=jax.ShapeDtypeStruct((N * C, HW), jnp.float32),
        mesh=mesh,
        scratch_types=scratch,
        compiler_params=pltpu.CompilerParams(
            use_tc_tiling_on_sc=False, needs_layout_passes=False),
    )
    def k(xt_hbm, gxy_hbm, out_hbm, gb_v, *rest):
        ii = [rest[0:4], rest[4:8]]
        ww = [rest[8:12], rest[12:16]]
        rr = [rest[16:20], rest[20:24]]
        ob = [rest[24], rest[25]]
        sems = [rest[26], rest[27]]
        osems = [rest[28], rest[29]]

        wid = lax.axis_index("s") * NC + lax.axis_index("c")
        px_base = wid * PXW
        batch = px_base // HW
        batch_off = batch * HW
        hw_base = px_base - batch_off
        cidx = [lax.iota(jnp.int32, L) + j * L for j in range(C // L)]

        def stage(g, s):
            pos = lax.rem(g, GB)

            @pl.when(pos == 0)
            def _():
                blk = px_base + g * P
                pltpu.sync_copy(gxy_hbm.at[pl.ds(blk, GBP)], gb_v.at[0])
                pltpu.sync_copy(gxy_hbm.at[pl.ds(NP + blk, GBP)], gb_v.at[1])

            off = pos * P
            for v in range(P // L):
                sl = pl.ds(off + v * L, L)
                so = pl.ds(v * L, L)
                gxv = gb_v[0, sl]
                gyv = gb_v[1, sl]
                ix = (gxv + 1.0) * (W * 0.5) - 0.5
                iy = (gyv + 1.0) * (H * 0.5) - 0.5
                tx = ix.astype(jnp.int32).astype(jnp.float32)
                ix0f = jnp.where(tx > ix, tx - 1.0, tx)
                ty = iy.astype(jnp.int32).astype(jnp.float32)
                iy0f = jnp.where(ty > iy, ty - 1.0, ty)
                wx1 = ix - ix0f
                wx0 = 1.0 - wx1
                wy1 = iy - iy0f
                wy0 = 1.0 - wy1
                ix0 = ix0f.astype(jnp.int32)
                ix1 = ix0 + 1
                iy0 = iy0f.astype(jnp.int32)
                iy1 = iy0 + 1
                vx0 = jnp.where((ix0 >= 0) & (ix0 < W), 1.0, 0.0)
                vx1 = jnp.where((ix1 >= 0) & (ix1 < W), 1.0, 0.0)
                vy0 = jnp.where((iy0 >= 0) & (iy0 < H), 1.0, 0.0)
                vy1 = jnp.where((iy1 >= 0) & (iy1 < H), 1.0, 0.0)
                xc0 = jnp.minimum(jnp.maximum(ix0, 0), W - 1)
                xc1 = jnp.minimum(jnp.maximum(ix1, 0), W - 1)
                yc0 = jnp.minimum(jnp.maximum(iy0, 0), H - 1)
                yc1 = jnp.minimum(jnp.maximum(iy1, 0), H - 1)
                r0 = yc0 * W + batch_off
                r1 = yc1 * W + batch_off
                ii[s][0][so] = r0 + xc0
                ii[s][1][so] = r0 + xc1
                ii[s][2][so] = r1 + xc0
                ii[s][3][so] = r1 + xc1
                ww[s][0][so] = wy0 * wx0 * vy0 * vx0
                ww[s][1][so] = wy0 * wx1 * vy0 * vx1
                ww[s][2][so] = wy1 * wx0 * vy1 * vx0
                ww[s][3][so] = wy1 * wx1 * vy1 * vx1
            for c in range(4):
                pltpu.async_copy(xt_hbm.at[ii[s][c]], rr[s][c], sems[s])

        def out_dst(g):
            return out_hbm.at[pl.ds(batch * C, C), pl.ds(hw_base + g * P, P)]

        def finish(g, s, first):
            for c in range(4):
                pltpu.make_async_copy(
                    xt_hbm.at[ii[s][c]], rr[s][c], sems[s]).wait()

            @pl.when(jnp.logical_not(first))
            def _():
                pltpu.make_async_copy(
                    ob[s].at[:, pl.ds(0, P)], out_dst(g), osems[s]).wait()

            r00_v, r01_v, r10_v, r11_v = rr[s]
            ob_v = ob[s]

            def grp_body(q, c2):
                qb = q * L
                sg = pl.ds(qb, L)
                wg00 = ww[s][0][sg]
                wg01 = ww[s][1][sg]
                wg10 = ww[s][2][sg]
                wg11 = ww[s][3][sg]
                for lane in range(L):
                    p = qb + lane
                    pv = lax.broadcast(p, (L,))
                    b00 = lax.broadcast(wg00[lane], (L,))
                    b01 = lax.broadcast(wg01[lane], (L,))
                    b10 = lax.broadcast(wg10[lane], (L,))
                    b11 = lax.broadcast(wg11[lane], (L,))
                    for j in range(C // L):
                        sj = pl.ds(j * L, L)
                        acc = (r00_v[p, sj] * b00 + r01_v[p, sj] * b01
                               + r10_v[p, sj] * b10 + r11_v[p, sj] * b11)
                        plsc.store_scatter(ob_v, [cidx[j], pv], acc)
                return c2

            lax.fori_loop(0, P // L, grp_body, 0, unroll=False)
            pltpu.async_copy(ob_v.at[:, pl.ds(0, P)], out_dst(g), osems[s])

        stage(0, 0)

        def body(t, carry):
            g0 = 2 * t
            stage(g0 + 1, 1)
            finish(g0, 0, t == 0)

            @pl.when(t < CHUNKS // 2 - 1)
            def _():
                stage(g0 + 2, 0)

            finish(g0 + 1, 1, t == 0)
            return carry

        lax.fori_loop(0, CHUNKS // 2, body, 0, unroll=False)
        for s in range(2):
            pltpu.make_async_copy(
                ob[s].at[:, pl.ds(0, P)], out_dst(CHUNKS - 2 + s),
                osems[s]).wait()

    return k(xt, gxy)


def kernel(x, grid):
    xt = jnp.pad(x.transpose(0, 2, 3, 1), ((0, 0), (0, 0), (0, 0), (0, CP - C))
                 ).reshape(NP, CP)
    gxy = grid.reshape(NP, 2).transpose(1, 0).reshape(2 * NP)
    out = _sc_grid_sample(xt, gxy)
    return out.reshape(N, C, H, W)


# speedup vs baseline: 1.3685x; 1.3685x over previous
import functools

import jax
import jax.numpy as jnp
from jax import lax
from jax.experimental import pallas as pl
from jax.experimental.pallas import tpu as pltpu
from jax.experimental.pallas import tpu_sc as plsc

N, C, H, W = 4, 96, 384, 384
CP = 128
HW = H * W
NP = N * HW
NC, NS, L = 2, 16, 16
NW = NC * NS
PXW = NP // NW
P = 64
CHUNKS = PXW // P
GB = 24
GBP = GB * P


def _sc_grid_sample(xt, gxy):
    mesh = plsc.VectorSubcoreMesh(
        core_axis_name="c", subcore_axis_name="s", num_cores=NC,
        num_subcores=NS)

    scratch = (
        [pltpu.VMEM((2, GBP), jnp.float32)]
        + [pltpu.VMEM((P,), jnp.int32)] * 8
        + [pltpu.VMEM((P,), jnp.float32)] * 8
        + [pltpu.VMEM((4 * P, CP), jnp.float32)] * 2
        + [pltpu.VMEM((P, CP), jnp.float32)] * 2
        + [pltpu.SemaphoreType.DMA] * 4
    )

    @functools.partial(
        pl.kernel,
        out_type=jax.ShapeDtypeStruct((NP, CP), jnp.float32),
        mesh=mesh,
        scratch_types=scratch,
        compiler_params=pltpu.CompilerParams(use_tc_tiling_on_sc=False),
    )
    def k(xt_hbm, gxy_hbm, out_hbm, gb_v, *rest):
        ii = [rest[0:4], rest[4:8]]
        ww = [rest[8:12], rest[12:16]]
        rr = [rest[16], rest[17]]
        ob = [rest[18], rest[19]]
        sems = [rest[20], rest[21]]
        osems = [rest[22], rest[23]]

        wid = lax.axis_index("s") * NC + lax.axis_index("c")
        px_base = wid * PXW
        batch_off = (px_base // HW) * HW

        zv = jnp.zeros((L,), jnp.float32)

        def zpad_body(p, c0):
            for s in range(2):
                ob[s][p, pl.ds(C, L)] = zv
                ob[s][p, pl.ds(C + L, L)] = zv
            return c0

        lax.fori_loop(0, P, zpad_body, 0, unroll=False)

        def stage(g, s):
            pos = lax.rem(g, GB)

            @pl.when(pos == 0)
            def _():
                blk = px_base + g * P
                pltpu.sync_copy(gxy_hbm.at[pl.ds(blk, GBP)], gb_v.at[0])
                pltpu.sync_copy(gxy_hbm.at[pl.ds(NP + blk, GBP)], gb_v.at[1])

            off = pos * P
            for v in range(P // L):
                sl = pl.ds(off + v * L, L)
                so = pl.ds(v * L, L)
                gxv = gb_v[0, sl]
                gyv = gb_v[1, sl]
                ix = (gxv + 1.0) * (W * 0.5) - 0.5
                iy = (gyv + 1.0) * (H * 0.5) - 0.5
                tx = ix.astype(jnp.int32).astype(jnp.float32)
                ix0f = jnp.where(tx > ix, tx - 1.0, tx)
                ty = iy.astype(jnp.int32).astype(jnp.float32)
                iy0f = jnp.where(ty > iy, ty - 1.0, ty)
                wx1 = ix - ix0f
                wx0 = 1.0 - wx1
                wy1 = iy - iy0f
                wy0 = 1.0 - wy1
                ix0 = ix0f.astype(jnp.int32)
                ix1 = ix0 + 1
                iy0 = iy0f.astype(jnp.int32)
                iy1 = iy0 + 1
                vx0 = jnp.where((ix0 >= 0) & (ix0 < W), 1.0, 0.0)
                vx1 = jnp.where((ix1 >= 0) & (ix1 < W), 1.0, 0.0)
                vy0 = jnp.where((iy0 >= 0) & (iy0 < H), 1.0, 0.0)
                vy1 = jnp.where((iy1 >= 0) & (iy1 < H), 1.0, 0.0)
                xc0 = jnp.minimum(jnp.maximum(ix0, 0), W - 1)
                xc1 = jnp.minimum(jnp.maximum(ix1, 0), W - 1)
                yc0 = jnp.minimum(jnp.maximum(iy0, 0), H - 1)
                yc1 = jnp.minimum(jnp.maximum(iy1, 0), H - 1)
                r0 = yc0 * W + batch_off
                r1 = yc1 * W + batch_off
                ii[s][0][so] = r0 + xc0
                ii[s][1][so] = r0 + xc1
                ii[s][2][so] = r1 + xc0
                ii[s][3][so] = r1 + xc1
                ww[s][0][so] = wy0 * wx0 * vy0 * vx0
                ww[s][1][so] = wy0 * wx1 * vy0 * vx1
                ww[s][2][so] = wy1 * wx0 * vy1 * vx0
                ww[s][3][so] = wy1 * wx1 * vy1 * vx1
            for c in range(4):
                pltpu.async_copy(
                    xt_hbm.at[ii[s][c]], rr[s].at[pl.ds(c * P, P)], sems[s])

        def out_dst(g):
            return out_hbm.at[pl.ds(px_base + g * P, P)]

        def finish(g, s, first):
            pltpu.make_async_copy(
                xt_hbm.at[pl.ds(0, 4 * P)], rr[s], sems[s]).wait()

            @pl.when(jnp.logical_not(first))
            def _():
                pltpu.make_async_copy(ob[s], out_dst(g), osems[s]).wait()

            rz_v = rr[s]
            ob_v = ob[s]

            def grp_body(q, c2):
                qb = q * L
                sg = pl.ds(qb, L)
                wg00 = ww[s][0][sg]
                wg01 = ww[s][1][sg]
                wg10 = ww[s][2][sg]
                wg11 = ww[s][3][sg]
                for lane in range(L):
                    p = qb + lane
                    b00 = lax.broadcast(wg00[lane], (L,))
                    b01 = lax.broadcast(wg01[lane], (L,))
                    b10 = lax.broadcast(wg10[lane], (L,))
                    b11 = lax.broadcast(wg11[lane], (L,))
                    for j in range(C // L):
                        sj = pl.ds(j * L, L)
                        ob_v[p, sj] = (
                            rz_v[p, sj] * b00 + rz_v[P + p, sj] * b01
                            + rz_v[2 * P + p, sj] * b10
                            + rz_v[3 * P + p, sj] * b11)
                return c2

            lax.fori_loop(0, P // L, grp_body, 0, unroll=False)
            pltpu.async_copy(ob_v, out_dst(g), osems[s])

        stage(0, 0)

        def body(t, carry):
            g0 = 2 * t
            stage(g0 + 1, 1)
            finish(g0, 0, t == 0)

            @pl.when(t < CHUNKS // 2 - 1)
            def _():
                stage(g0 + 2, 0)

            finish(g0 + 1, 1, t == 0)
            return carry

        lax.fori_loop(0, CHUNKS // 2, body, 0, unroll=False)
        for s in range(2):
            pltpu.make_async_copy(
                ob[s], out_dst(CHUNKS - 2 + s), osems[s]).wait()

    return k(xt, gxy)


def kernel(x, grid):
    xt = jnp.pad(x.transpose(0, 2, 3, 1), ((0, 0), (0, 0), (0, 0), (0, CP - C))
                 ).reshape(NP, CP)
    gxy = grid.reshape(NP, 2).transpose(1, 0).reshape(2 * NP)
    out = _sc_grid_sample(xt, gxy)
    return out[:, :C].reshape(N, H, W, C).transpose(0, 3, 1, 2)
